# baseline (speedup 1.0000x reference)
import functools
import jax
import jax.numpy as jnp
from jax import lax
from jax.experimental import pallas as pl
from jax.experimental.pallas import tpu as pltpu
from jax.experimental.pallas import tpu_sc as plsc

N = 10000
E = 320000
D = 128
NC = 2
NS = 16
NW = NC * NS
NP = 10240
DK = 80
DC = E // NW // DK
AK = 25
AB = 4
AC = E // NW // AK // AB
NB = 4
RPT = NP // NS
TILE = 400
GRID = N // TILE


def _mesh():
    return plsc.VectorSubcoreMesh(
        core_axis_name="c", subcore_axis_name="s", num_cores=NC, num_subcores=NS
    )


def _deg_body(dst_hbm, ones_hbm, z_hbm, out_hbm, didx, ones_v, acc, sem):
    cid = lax.axis_index("c")
    sid = lax.axis_index("s")
    wid = cid * NS + sid
    pltpu.sync_copy(ones_hbm, ones_v)
    pltpu.sync_copy(dst_hbm.at[wid], didx)
    pltpu.sync_copy(z_hbm, acc.at[pl.ds(sid * RPT, RPT)])
    plsc.subcore_barrier()

    def step(j, carry):
        pltpu.sync_copy(ones_v, acc.at[didx.at[j]], add=True)
        return carry

    lax.fori_loop(0, DC, step, 0)
    plsc.subcore_barrier()
    pltpu.sync_copy(
        acc.at[pl.ds(sid * RPT, RPT)], out_hbm.at[cid, pl.ds(sid * RPT, RPT)]
    )


def _degrees(dst2d, ones16, z16):
    return pl.kernel(
        _deg_body,
        out_type=jax.ShapeDtypeStruct((NC, NP), jnp.float32),
        mesh=_mesh(),
        scratch_types=[
            pltpu.VMEM((DC, DK), jnp.int32),
            pltpu.VMEM((DK,), jnp.float32),
            pltpu.VMEM_SHARED((NP,), jnp.float32),
            pltpu.SemaphoreType.DMA,
        ],
    )(dst2d, ones16, z16)


def _agg_body(hp_hbm, src_hbm, dst_hbm, z_hbm, out_hbm,
              sidx, didx, r0, r1, r2, r3, acc,
              g0, g1, g2, g3, s0, s1, s2, s3):
    rows = (r0, r1, r2, r3)
    gs = (g0, g1, g2, g3)
    ss = (s0, s1, s2, s3)
    cid = lax.axis_index("c")
    sid = lax.axis_index("s")
    wid = cid * NS + sid
    pltpu.sync_copy(z_hbm, acc.at[pl.ds(sid * RPT, RPT)])
    plsc.subcore_barrier()

    def block(b, carry):
        pltpu.sync_copy(src_hbm.at[wid, b], sidx)
        pltpu.sync_copy(dst_hbm.at[wid, b], didx)
        pltpu.async_copy(hp_hbm.at[sidx.at[0]], rows[0], gs[0])
        pltpu.async_copy(hp_hbm.at[sidx.at[1]], rows[1], gs[1])

        def quad(i, c2):
            for u in range(NB):
                j = NB * i + u
                pltpu.make_async_copy(hp_hbm.at[sidx.at[j]], rows[u], gs[u]).wait()
                pltpu.async_copy(rows[u], acc.at[didx.at[j]], ss[u], add=True)
                un = (u + 2) % NB

                @pl.when(j + 2 < AC)
                def _():
                    @pl.when(j >= 2)
                    def _():
                        pltpu.make_async_copy(
                            rows[un], acc.at[didx.at[j - 2]], ss[un]
                        ).wait()

                    pltpu.async_copy(hp_hbm.at[sidx.at[j + 2]], rows[un], gs[un])

            return c2

        lax.fori_loop(0, AC // NB, quad, carry)
        for u in range(NB):
            pltpu.make_async_copy(rows[u], acc.at[didx.at[AC - NB + u]], ss[u]).wait()
        return carry

    lax.fori_loop(0, AB, block, 0)
    plsc.subcore_barrier()
    pltpu.sync_copy(
        acc.at[pl.ds(sid * RPT, RPT)], out_hbm.at[cid, pl.ds(sid * RPT, RPT)]
    )


def _aggregate(hp, src4d, dst4d, z128):
    return pl.kernel(
        _agg_body,
        out_type=jax.ShapeDtypeStruct((NC, NP, D), jnp.float32),
        mesh=_mesh(),
        scratch_types=[
            pltpu.VMEM((AC, AK), jnp.int32),
            pltpu.VMEM((AC, AK), jnp.int32),
            pltpu.VMEM((AK, D), jnp.float32),
            pltpu.VMEM((AK, D), jnp.float32),
            pltpu.VMEM((AK, D), jnp.float32),
            pltpu.VMEM((AK, D), jnp.float32),
            pltpu.VMEM_SHARED((NP, D), jnp.float32),
            pltpu.SemaphoreType.DMA,
            pltpu.SemaphoreType.DMA,
            pltpu.SemaphoreType.DMA,
            pltpu.SemaphoreType.DMA,
            pltpu.SemaphoreType.DMA,
            pltpu.SemaphoreType.DMA,
            pltpu.SemaphoreType.DMA,
            pltpu.SemaphoreType.DMA,
        ],
    )(hp, src4d, dst4d, z128)


def _tc1_body(deg_ref, x_ref, w_ref, dis_ref, hp_ref):
    deg = deg_ref[0, :, 0] + deg_ref[1, :, 0] + 1.0
    dis = lax.rsqrt(deg)
    h = jnp.dot(x_ref[...], w_ref[...], preferred_element_type=jnp.float32)
    hp_ref[...] = h * dis[:, None]
    dis_ref[...] = jnp.broadcast_to(dis[:, None], (TILE, 8))


def _tc1(degp, x, w1):
    return pl.pallas_call(
        _tc1_body,
        grid=(GRID,),
        in_specs=[
            pl.BlockSpec((NC, TILE, 1), lambda i: (0, i, 0)),
            pl.BlockSpec((TILE, D), lambda i: (i, 0)),
            pl.BlockSpec((D, D), lambda i: (0, 0)),
        ],
        out_specs=[
            pl.BlockSpec((TILE, 8), lambda i: (i, 0)),
            pl.BlockSpec((TILE, D), lambda i: (i, 0)),
        ],
        out_shape=[
            jax.ShapeDtypeStruct((N, 8), jnp.float32),
            jax.ShapeDtypeStruct((N, D), jnp.float32),
        ],
    )(degp, x, w1)


def _tc2_body(agg_ref, hp_ref, dis_ref, b_ref, w_ref, out_ref):
    dis = dis_ref[:, 0][:, None]
    t = (agg_ref[0] + agg_ref[1] + hp_ref[...]) * dis + b_ref[...]
    h = jnp.maximum(t, 0.0)
    out_ref[...] = jnp.dot(h, w_ref[...], preferred_element_type=jnp.float32) * dis


def _tc2(agg, hp, dis8, b1, w2):
    return pl.pallas_call(
        _tc2_body,
        grid=(GRID,),
        in_specs=[
            pl.BlockSpec((NC, TILE, D), lambda i: (0, i, 0)),
            pl.BlockSpec((TILE, D), lambda i: (i, 0)),
            pl.BlockSpec((TILE, 8), lambda i: (i, 0)),
            pl.BlockSpec((1, D), lambda i: (0, 0)),
            pl.BlockSpec((D, D), lambda i: (0, 0)),
        ],
        out_specs=pl.BlockSpec((TILE, D), lambda i: (i, 0)),
        out_shape=jax.ShapeDtypeStruct((N, D), jnp.float32),
    )(agg, hp, dis8, b1, w2)


def _tc3_body(agg_ref, hp_ref, dis_ref, b_ref, w_ref, b3_ref, out_ref):
    dis = dis_ref[:, 0][:, None]
    t = (agg_ref[0] + agg_ref[1] + hp_ref[...]) * dis + b_ref[...]
    h = jnp.maximum(t, 0.0)
    out_ref[...] = (
        jnp.dot(h, w_ref[...], preferred_element_type=jnp.float32) + b3_ref[...]
    )


def _tc3(agg, hp, dis8, b2, w3p, b3p):
    return pl.pallas_call(
        _tc3_body,
        grid=(GRID,),
        in_specs=[
            pl.BlockSpec((NC, TILE, D), lambda i: (0, i, 0)),
            pl.BlockSpec((TILE, D), lambda i: (i, 0)),
            pl.BlockSpec((TILE, 8), lambda i: (i, 0)),
            pl.BlockSpec((1, D), lambda i: (0, 0)),
            pl.BlockSpec((D, D), lambda i: (0, 0)),
            pl.BlockSpec((1, D), lambda i: (0, 0)),
        ],
        out_specs=pl.BlockSpec((TILE, D), lambda i: (i, 0)),
        out_shape=jax.ShapeDtypeStruct((N, D), jnp.float32),
    )(agg, hp, dis8, b2, w3p, b3p)


def kernel(x, edge_index, W1, b1, W2, b2, W3, b3):
    srcd = edge_index[0].reshape(NW, DC, DK)
    dstd = edge_index[1].reshape(NW, DC, DK)
    src4d = edge_index[0].reshape(NW, AB, AC, AK)
    dst4d = edge_index[1].reshape(NW, AB, AC, AK)
    z128 = jnp.zeros((RPT, D), jnp.float32)
    z1 = jnp.zeros((RPT,), jnp.float32)
    ones1 = jnp.ones((DK,), jnp.float32)

    degp = _degrees(dstd, ones1, z1).reshape(NC, NP, 1)
    dis8, hp1 = _tc1(degp, x, W1)
    agg1 = _aggregate(hp1, src4d, dst4d, z128)
    hp2 = _tc2(agg1, hp1, dis8, b1.reshape(1, D), W2)
    agg2 = _aggregate(hp2, src4d, dst4d, z128)
    w3p = jnp.pad(W3, ((0, 0), (0, D - W3.shape[1])))
    b3p = jnp.pad(b3, (0, D - b3.shape[0])).reshape(1, D)
    out = _tc3(agg2, hp2, dis8, b2.reshape(1, D), w3p, b3p)
    return out[:, : W3.shape[1]]


# speedup vs baseline: 1.2934x; 1.2934x over previous
import functools
import jax
import jax.numpy as jnp
from jax import lax
from jax.experimental import pallas as pl
from jax.experimental.pallas import tpu as pltpu
from jax.experimental.pallas import tpu_sc as plsc

N = 10000
E = 320000
D = 128
NC = 2
NS = 16
NW = NC * NS
NP = 10240
DK = 80
DC = E // NW // DK
AK = 80
AB = 5
AC = E // NW // AK // AB
RPT = NP // NS
TILE = 400
GRID = N // TILE


def _mesh():
    return plsc.VectorSubcoreMesh(
        core_axis_name="c", subcore_axis_name="s", num_cores=NC, num_subcores=NS
    )


def _deg_body(dst_hbm, ones_hbm, z_hbm, out_hbm, didx, ones_v, acc, sem):
    cid = lax.axis_index("c")
    sid = lax.axis_index("s")
    wid = cid * NS + sid
    pltpu.sync_copy(ones_hbm, ones_v)
    pltpu.sync_copy(dst_hbm.at[wid], didx)
    pltpu.sync_copy(z_hbm, acc.at[pl.ds(sid * RPT, RPT)])
    plsc.subcore_barrier()

    def step(j, carry):
        pltpu.sync_copy(ones_v, acc.at[didx.at[j]], add=True)
        return carry

    lax.fori_loop(0, DC, step, 0)
    plsc.subcore_barrier()
    pltpu.sync_copy(
        acc.at[pl.ds(sid * RPT, RPT)], out_hbm.at[cid, pl.ds(sid * RPT, RPT)]
    )


def _degrees(dst2d, ones16, z16):
    return pl.kernel(
        _deg_body,
        out_type=jax.ShapeDtypeStruct((NC, NP), jnp.float32),
        mesh=_mesh(),
        scratch_types=[
            pltpu.VMEM((DC, DK), jnp.int32),
            pltpu.VMEM((DK,), jnp.float32),
            pltpu.VMEM_SHARED((NP,), jnp.float32),
            pltpu.SemaphoreType.DMA,
        ],
    )(dst2d, ones16, z16)


def _agg_body(hp_hbm, src_hbm, dst_hbm, z_hbm, out_hbm,
              sidx, didx, rows0, rows1, acc, gsem0, gsem1):
    cid = lax.axis_index("c")
    sid = lax.axis_index("s")
    wid = cid * NS + sid
    pltpu.sync_copy(z_hbm, acc.at[pl.ds(sid * RPT, RPT)])
    plsc.subcore_barrier()

    def block(b, carry):
        pltpu.sync_copy(src_hbm.at[wid, b], sidx)
        pltpu.sync_copy(dst_hbm.at[wid, b], didx)
        pltpu.async_copy(hp_hbm.at[sidx.at[0]], rows0, gsem0)
        pltpu.async_copy(hp_hbm.at[sidx.at[1]], rows1, gsem1)

        def pair(i, c2):
            j0 = 2 * i
            j1 = 2 * i + 1
            pltpu.make_async_copy(hp_hbm.at[sidx.at[j0]], rows0, gsem0).wait()
            pltpu.sync_copy(rows0, acc.at[didx.at[j0]], add=True)

            @pl.when(j0 + 2 < AC)
            def _():
                pltpu.async_copy(hp_hbm.at[sidx.at[j0 + 2]], rows0, gsem0)

            pltpu.make_async_copy(hp_hbm.at[sidx.at[j1]], rows1, gsem1).wait()
            pltpu.sync_copy(rows1, acc.at[didx.at[j1]], add=True)

            @pl.when(j1 + 2 < AC)
            def _():
                pltpu.async_copy(hp_hbm.at[sidx.at[j1 + 2]], rows1, gsem1)

            return c2

        lax.fori_loop(0, AC // 2, pair, carry)
        if AC % 2:
            jt = AC - 1
            pltpu.make_async_copy(hp_hbm.at[sidx.at[jt]], rows0, gsem0).wait()
            pltpu.sync_copy(rows0, acc.at[didx.at[jt]], add=True)
        return carry

    lax.fori_loop(0, AB, block, 0)
    plsc.subcore_barrier()
    pltpu.sync_copy(
        acc.at[pl.ds(sid * RPT, RPT)], out_hbm.at[cid, pl.ds(sid * RPT, RPT)]
    )


def _aggregate(hp, src4d, dst4d, z128):
    return pl.kernel(
        _agg_body,
        out_type=jax.ShapeDtypeStruct((NC, NP, D), jnp.float32),
        mesh=_mesh(),
        scratch_types=[
            pltpu.VMEM((AC, AK), jnp.int32),
            pltpu.VMEM((AC, AK), jnp.int32),
            pltpu.VMEM((AK, D), jnp.float32),
            pltpu.VMEM((AK, D), jnp.float32),
            pltpu.VMEM_SHARED((NP, D), jnp.float32),
            pltpu.SemaphoreType.DMA,
            pltpu.SemaphoreType.DMA,
        ],
    )(hp, src4d, dst4d, z128)


def _tc1_body(deg_ref, x_ref, w_ref, dis_ref, hp_ref):
    deg = deg_ref[0, :, 0] + deg_ref[1, :, 0] + 1.0
    dis = lax.rsqrt(deg)
    h = jnp.dot(x_ref[...], w_ref[...], preferred_element_type=jnp.float32)
    hp_ref[...] = h * dis[:, None]
    dis_ref[...] = jnp.broadcast_to(dis[:, None], (TILE, 8))


def _tc1(degp, x, w1):
    return pl.pallas_call(
        _tc1_body,
        grid=(GRID,),
        in_specs=[
            pl.BlockSpec((NC, TILE, 1), lambda i: (0, i, 0)),
            pl.BlockSpec((TILE, D), lambda i: (i, 0)),
            pl.BlockSpec((D, D), lambda i: (0, 0)),
        ],
        out_specs=[
            pl.BlockSpec((TILE, 8), lambda i: (i, 0)),
            pl.BlockSpec((TILE, D), lambda i: (i, 0)),
        ],
        out_shape=[
            jax.ShapeDtypeStruct((N, 8), jnp.float32),
            jax.ShapeDtypeStruct((N, D), jnp.float32),
        ],
    )(degp, x, w1)


def _tc2_body(agg_ref, hp_ref, dis_ref, b_ref, w_ref, out_ref):
    dis = dis_ref[:, 0][:, None]
    t = (agg_ref[0] + agg_ref[1] + hp_ref[...]) * dis + b_ref[...]
    h = jnp.maximum(t, 0.0)
    out_ref[...] = jnp.dot(h, w_ref[...], preferred_element_type=jnp.float32) * dis


def _tc2(agg, hp, dis8, b1, w2):
    return pl.pallas_call(
        _tc2_body,
        grid=(GRID,),
        in_specs=[
            pl.BlockSpec((NC, TILE, D), lambda i: (0, i, 0)),
            pl.BlockSpec((TILE, D), lambda i: (i, 0)),
            pl.BlockSpec((TILE, 8), lambda i: (i, 0)),
            pl.BlockSpec((1, D), lambda i: (0, 0)),
            pl.BlockSpec((D, D), lambda i: (0, 0)),
        ],
        out_specs=pl.BlockSpec((TILE, D), lambda i: (i, 0)),
        out_shape=jax.ShapeDtypeStruct((N, D), jnp.float32),
    )(agg, hp, dis8, b1, w2)


def _tc3_body(agg_ref, hp_ref, dis_ref, b_ref, w_ref, b3_ref, out_ref):
    dis = dis_ref[:, 0][:, None]
    t = (agg_ref[0] + agg_ref[1] + hp_ref[...]) * dis + b_ref[...]
    h = jnp.maximum(t, 0.0)
    out_ref[...] = (
        jnp.dot(h, w_ref[...], preferred_element_type=jnp.float32) + b3_ref[...]
    )


def _tc3(agg, hp, dis8, b2, w3p, b3p):
    return pl.pallas_call(
        _tc3_body,
        grid=(GRID,),
        in_specs=[
            pl.BlockSpec((NC, TILE, D), lambda i: (0, i, 0)),
            pl.BlockSpec((TILE, D), lambda i: (i, 0)),
            pl.BlockSpec((TILE, 8), lambda i: (i, 0)),
            pl.BlockSpec((1, D), lambda i: (0, 0)),
            pl.BlockSpec((D, D), lambda i: (0, 0)),
            pl.BlockSpec((1, D), lambda i: (0, 0)),
        ],
        out_specs=pl.BlockSpec((TILE, D), lambda i: (i, 0)),
        out_shape=jax.ShapeDtypeStruct((N, D), jnp.float32),
    )(agg, hp, dis8, b2, w3p, b3p)


def kernel(x, edge_index, W1, b1, W2, b2, W3, b3):
    srcd = edge_index[0].reshape(NW, DC, DK)
    dstd = edge_index[1].reshape(NW, DC, DK)
    src4d = edge_index[0].reshape(NW, AB, AC, AK)
    dst4d = edge_index[1].reshape(NW, AB, AC, AK)
    z128 = jnp.zeros((RPT, D), jnp.float32)
    z1 = jnp.zeros((RPT,), jnp.float32)
    ones1 = jnp.ones((DK,), jnp.float32)

    degp = _degrees(dstd, ones1, z1).reshape(NC, NP, 1)
    dis8, hp1 = _tc1(degp, x, W1)
    agg1 = _aggregate(hp1, src4d, dst4d, z128)
    hp2 = _tc2(agg1, hp1, dis8, b1.reshape(1, D), W2)
    agg2 = _aggregate(hp2, src4d, dst4d, z128)
    w3p = jnp.pad(W3, ((0, 0), (0, D - W3.shape[1])))
    b3p = jnp.pad(b3, (0, D - b3.shape[0])).reshape(1, D)
    out = _tc3(agg2, hp2, dis8, b2.reshape(1, D), w3p, b3p)
    return out[:, : W3.shape[1]]


# speedup vs baseline: 1.4048x; 1.0862x over previous
import functools
import jax
import jax.numpy as jnp
from jax import lax
from jax.experimental import pallas as pl
from jax.experimental.pallas import tpu as pltpu
from jax.experimental.pallas import tpu_sc as plsc

N = 10000
E = 320000
D = 128
NC = 2
NS = 16
NW = NC * NS
NP = 10240
DK = 80
DC = E // NW // DK
AK = 80
AB = 5
AC = E // NW // AK // AB
RPT = NP // NS
TILE = 2000
GRID = N // TILE


def _mesh():
    return plsc.VectorSubcoreMesh(
        core_axis_name="c", subcore_axis_name="s", num_cores=NC, num_subcores=NS
    )


def _deg_body(dst_hbm, ones_hbm, z_hbm, out_hbm, didx, ones_v, acc, sem):
    cid = lax.axis_index("c")
    sid = lax.axis_index("s")
    wid = cid * NS + sid
    pltpu.sync_copy(ones_hbm, ones_v)
    pltpu.sync_copy(dst_hbm.at[wid], didx)
    pltpu.sync_copy(z_hbm, acc.at[pl.ds(sid * RPT, RPT)])
    plsc.subcore_barrier()

    def step(b, carry):
        def inner(j, c2):
            pltpu.sync_copy(ones_v, acc.at[didx.at[b, j]], add=True)
            return c2
        return lax.fori_loop(0, AC, inner, carry)

    lax.fori_loop(0, AB, step, 0)
    plsc.subcore_barrier()
    pltpu.sync_copy(
        acc.at[pl.ds(sid * RPT, RPT)], out_hbm.at[cid, pl.ds(sid * RPT, RPT)]
    )


def _degrees(dst2d, ones16, z16):
    return pl.kernel(
        _deg_body,
        out_type=jax.ShapeDtypeStruct((NC, NP), jnp.float32),
        mesh=_mesh(),
        scratch_types=[
            pltpu.VMEM((AB, AC, AK), jnp.int32),
            pltpu.VMEM((AK,), jnp.float32),
            pltpu.VMEM_SHARED((NP,), jnp.float32),
            pltpu.SemaphoreType.DMA,
        ],
    )(dst2d, ones16, z16)


def _agg_body(hp_hbm, src_hbm, dst_hbm, z_hbm, out_hbm,
              sidx, didx, rows0, rows1, acc, gsem0, gsem1):
    cid = lax.axis_index("c")
    sid = lax.axis_index("s")
    wid = cid * NS + sid
    pltpu.sync_copy(z_hbm, acc.at[pl.ds(sid * RPT, RPT)])
    plsc.subcore_barrier()

    def block(b, carry):
        pltpu.sync_copy(src_hbm.at[wid, b], sidx)
        pltpu.sync_copy(dst_hbm.at[wid, b], didx)
        pltpu.async_copy(hp_hbm.at[sidx.at[0]], rows0, gsem0)
        pltpu.async_copy(hp_hbm.at[sidx.at[1]], rows1, gsem1)

        def pair(i, c2):
            j0 = 2 * i
            j1 = 2 * i + 1
            pltpu.make_async_copy(hp_hbm.at[sidx.at[j0]], rows0, gsem0).wait()
            pltpu.sync_copy(rows0, acc.at[didx.at[j0]], add=True)

            @pl.when(j0 + 2 < AC)
            def _():
                pltpu.async_copy(hp_hbm.at[sidx.at[j0 + 2]], rows0, gsem0)

            pltpu.make_async_copy(hp_hbm.at[sidx.at[j1]], rows1, gsem1).wait()
            pltpu.sync_copy(rows1, acc.at[didx.at[j1]], add=True)

            @pl.when(j1 + 2 < AC)
            def _():
                pltpu.async_copy(hp_hbm.at[sidx.at[j1 + 2]], rows1, gsem1)

            return c2

        lax.fori_loop(0, AC // 2, pair, carry)
        if AC % 2:
            jt = AC - 1
            pltpu.make_async_copy(hp_hbm.at[sidx.at[jt]], rows0, gsem0).wait()
            pltpu.sync_copy(rows0, acc.at[didx.at[jt]], add=True)
        return carry

    lax.fori_loop(0, AB, block, 0)
    plsc.subcore_barrier()
    pltpu.sync_copy(
        acc.at[pl.ds(sid * RPT, RPT)], out_hbm.at[cid, pl.ds(sid * RPT, RPT)]
    )


def _aggregate(hp, src4d, dst4d, z128):
    return pl.kernel(
        _agg_body,
        out_type=jax.ShapeDtypeStruct((NC, NP, D), jnp.float32),
        mesh=_mesh(),
        scratch_types=[
            pltpu.VMEM((AC, AK), jnp.int32),
            pltpu.VMEM((AC, AK), jnp.int32),
            pltpu.VMEM((AK, D), jnp.float32),
            pltpu.VMEM((AK, D), jnp.float32),
            pltpu.VMEM_SHARED((NP, D), jnp.float32),
            pltpu.SemaphoreType.DMA,
            pltpu.SemaphoreType.DMA,
        ],
    )(hp, src4d, dst4d, z128)


def _tc1_body(deg_ref, x_ref, w_ref, dis_ref, hp_ref):
    deg = deg_ref[0, :, 0] + deg_ref[1, :, 0] + 1.0
    dis = lax.rsqrt(deg)
    h = jnp.dot(x_ref[...], w_ref[...], preferred_element_type=jnp.float32)
    hp_ref[...] = h * dis[:, None]
    dis_ref[...] = jnp.broadcast_to(dis[:, None], (TILE, 8))


def _tc1(degp, x, w1):
    return pl.pallas_call(
        _tc1_body,
        grid=(GRID,),
        in_specs=[
            pl.BlockSpec((NC, TILE, 1), lambda i: (0, i, 0)),
            pl.BlockSpec((TILE, D), lambda i: (i, 0)),
            pl.BlockSpec((D, D), lambda i: (0, 0)),
        ],
        out_specs=[
            pl.BlockSpec((TILE, 8), lambda i: (i, 0)),
            pl.BlockSpec((TILE, D), lambda i: (i, 0)),
        ],
        out_shape=[
            jax.ShapeDtypeStruct((N, 8), jnp.float32),
            jax.ShapeDtypeStruct((N, D), jnp.float32),
        ],
    )(degp, x, w1)


def _tc2_body(agg_ref, hp_ref, dis_ref, b_ref, w_ref, out_ref):
    dis = dis_ref[:, 0][:, None]
    t = (agg_ref[0] + agg_ref[1] + hp_ref[...]) * dis + b_ref[...]
    h = jnp.maximum(t, 0.0)
    out_ref[...] = jnp.dot(h, w_ref[...], preferred_element_type=jnp.float32) * dis


def _tc2(agg, hp, dis8, b1, w2):
    return pl.pallas_call(
        _tc2_body,
        grid=(GRID,),
        in_specs=[
            pl.BlockSpec((NC, TILE, D), lambda i: (0, i, 0)),
            pl.BlockSpec((TILE, D), lambda i: (i, 0)),
            pl.BlockSpec((TILE, 8), lambda i: (i, 0)),
            pl.BlockSpec((1, D), lambda i: (0, 0)),
            pl.BlockSpec((D, D), lambda i: (0, 0)),
        ],
        out_specs=pl.BlockSpec((TILE, D), lambda i: (i, 0)),
        out_shape=jax.ShapeDtypeStruct((N, D), jnp.float32),
    )(agg, hp, dis8, b1, w2)


def _tc3_body(agg_ref, hp_ref, dis_ref, b_ref, w_ref, b3_ref, out_ref):
    dis = dis_ref[:, 0][:, None]
    t = (agg_ref[0] + agg_ref[1] + hp_ref[...]) * dis + b_ref[...]
    h = jnp.maximum(t, 0.0)
    out_ref[...] = (
        jnp.dot(h, w_ref[...], preferred_element_type=jnp.float32) + b3_ref[...]
    )


def _tc3(agg, hp, dis8, b2, w3p, b3p):
    return pl.pallas_call(
        _tc3_body,
        grid=(GRID,),
        in_specs=[
            pl.BlockSpec((NC, TILE, D), lambda i: (0, i, 0)),
            pl.BlockSpec((TILE, D), lambda i: (i, 0)),
            pl.BlockSpec((TILE, 8), lambda i: (i, 0)),
            pl.BlockSpec((1, D), lambda i: (0, 0)),
            pl.BlockSpec((D, 8), lambda i: (0, 0)),
            pl.BlockSpec((1, 8), lambda i: (0, 0)),
        ],
        out_specs=pl.BlockSpec((TILE, 8), lambda i: (i, 0)),
        out_shape=jax.ShapeDtypeStruct((N, 8), jnp.float32),
    )(agg, hp, dis8, b2, w3p, b3p)


def kernel(x, edge_index, W1, b1, W2, b2, W3, b3):
    src4d = edge_index[0].reshape(NW, AB, AC, AK)
    dst4d = edge_index[1].reshape(NW, AB, AC, AK)
    z128 = jnp.zeros((RPT, D), jnp.float32)
    z1 = jnp.zeros((RPT,), jnp.float32)
    ones1 = jnp.ones((AK,), jnp.float32)

    degp = _degrees(dst4d, ones1, z1).reshape(NC, NP, 1)
    dis8, hp1 = _tc1(degp, x, W1)
    agg1 = _aggregate(hp1, src4d, dst4d, z128)
    hp2 = _tc2(agg1, hp1, dis8, b1.reshape(1, D), W2)
    agg2 = _aggregate(hp2, src4d, dst4d, z128)
    w3p = jnp.pad(W3, ((0, 0), (0, 8 - W3.shape[1])))
    b3p = jnp.pad(b3, (0, 8 - b3.shape[0])).reshape(1, 8)
    out = _tc3(agg2, hp2, dis8, b2.reshape(1, D), w3p, b3p)
    return out[:, : W3.shape[1]]
